# baseline (speedup 1.0000x reference)
import jax
import jax.numpy as jnp
from jax import lax
from jax.experimental import pallas as pl
from jax.experimental.pallas import tpu as pltpu
from jax.experimental.pallas import tpu_sc as plsc

_B = 16384
_D = 16
_NC = 2
_NS = 16
_NW = _NC * _NS
_BPW = _B // _NW
_CHUNK = 128
_NCH = _BPW // _CHUNK
_GROUPS = _BPW // 16


def _bpr_body(uidx_hbm, iidx_hbm, uemb_hbm, iemb_hbm, out_hbm,
              uidx_v, iidx_v, urows_v, irows_v, out_v, usem, isem):
    wid = lax.axis_index("s") * _NC + lax.axis_index("c")
    row0 = wid * _NCH
    base = wid * _BPW

    pltpu.sync_copy(uidx_hbm.at[pl.ds(row0, _NCH)], uidx_v)
    pltpu.sync_copy(iidx_hbm.at[pl.ds(row0, _NCH)], iidx_v)

    copies = []
    for j in range(_NCH):
        copies.append(pltpu.async_copy(
            uemb_hbm.at[uidx_v.at[j]],
            urows_v.at[pl.ds(j * _CHUNK, _CHUNK)], usem))
        copies.append(pltpu.async_copy(
            iemb_hbm.at[iidx_v.at[j]],
            irows_v.at[pl.ds(j * _CHUNK, _CHUNK)], isem))
    for c in copies:
        c.wait()

    lanes = lax.iota(jnp.int32, 16)

    def group_body(g, carry):
        rows = g * 16 + lanes
        acc = jnp.zeros((16,), jnp.float32)
        for d in range(_D):
            cols = jnp.full((16,), d, jnp.int32)
            u = plsc.load_gather(urows_v, [rows, cols])
            v = plsc.load_gather(irows_v, [rows, cols])
            acc = acc + u * v
        out_v[pl.ds(g * 16, 16)] = acc
        return carry

    lax.fori_loop(0, _GROUPS, group_body, 0)

    pltpu.sync_copy(out_v, out_hbm.at[pl.ds(base, _BPW)])


def kernel(user_idx, item_idx, user_emb, item_emb):
    uidx2 = user_idx.reshape(_NW * _NCH, _CHUNK)
    iidx2 = item_idx.reshape(_NW * _NCH, _CHUNK)
    mesh = plsc.VectorSubcoreMesh(core_axis_name="c", subcore_axis_name="s")
    f = pl.kernel(
        _bpr_body,
        out_type=jax.ShapeDtypeStruct((_B,), jnp.float32),
        mesh=mesh,
        compiler_params=pltpu.CompilerParams(
            needs_layout_passes=False, use_tc_tiling_on_sc=False),
        scratch_types=[
            pltpu.VMEM((_NCH, _CHUNK), jnp.int32),
            pltpu.VMEM((_NCH, _CHUNK), jnp.int32),
            pltpu.VMEM((_BPW, _D), jnp.float32),
            pltpu.VMEM((_BPW, _D), jnp.float32),
            pltpu.VMEM((_BPW,), jnp.float32),
            pltpu.SemaphoreType.DMA,
            pltpu.SemaphoreType.DMA,
        ],
    )
    return f(uidx2, iidx2, user_emb, item_emb)


# speedup vs baseline: 1.3454x; 1.3454x over previous
import jax
import jax.numpy as jnp
from jax import lax
from jax.experimental import pallas as pl
from jax.experimental.pallas import tpu as pltpu
from jax.experimental.pallas import tpu_sc as plsc

_B = 16384
_D = 16
_NC = 2
_NS = 16
_NW = _NC * _NS
_BPW = _B // _NW
_PASS = 32
_NP = _BPW // _PASS


def _bpr_body(uidx_hbm, iidx_hbm, uemb_hbm, iemb_hbm, out_hbm,
              uidx_v, iidx_v, ublk_v, iblk_v, out_v,
              usem, isem):
    wid = lax.axis_index("s") * _NC + lax.axis_index("c")
    base = wid * _BPW

    pltpu.sync_copy(uidx_hbm.at[pl.ds(base, _BPW)], uidx_v)
    pltpu.sync_copy(iidx_hbm.at[pl.ds(base, _BPW)], iidx_v)

    lanes = lax.iota(jnp.int32, 16)

    for p in range(_NP):
        def fire_body(g, carry):
            uv = uidx_v[pl.ds(p * _PASS + g * 16, 16)]
            iv = iidx_v[pl.ds(p * _PASS + g * 16, 16)]
            for l in range(16):
                us = uv[l]
                is_ = iv[l]
                ublk = pl.multiple_of((us >> 3) * 8, 8)
                iblk = pl.multiple_of((is_ >> 3) * 8, 8)
                i = g * 16 + l
                pltpu.async_copy(uemb_hbm.at[pl.ds(ublk, 8)],
                                 ublk_v.at[pl.ds(i * 8, 8)], usem)
                pltpu.async_copy(iemb_hbm.at[pl.ds(iblk, 8)],
                                 iblk_v.at[pl.ds(i * 8, 8)], isem)
            return carry

        lax.fori_loop(0, _PASS // 16, fire_body, 0)

        pltpu.make_async_copy(uemb_hbm.at[pl.ds(0, 8 * _PASS)],
                              ublk_v, usem).wait()
        pltpu.make_async_copy(iemb_hbm.at[pl.ds(0, 8 * _PASS)],
                              iblk_v, isem).wait()

        def group_body(g, carry):
            rows = g * 16 + lanes
            uv = uidx_v[pl.ds(p * _PASS + g * 16, 16)]
            iv = iidx_v[pl.ds(p * _PASS + g * 16, 16)]
            usub = jnp.bitwise_and(uv, 7)
            isub = jnp.bitwise_and(iv, 7)
            urow = rows * 8 + usub
            irow = rows * 8 + isub
            acc = jnp.zeros((16,), jnp.float32)
            for d in range(_D):
                cols = jnp.full((16,), d, jnp.int32)
                u = plsc.load_gather(ublk_v, [urow, cols])
                v = plsc.load_gather(iblk_v, [irow, cols])
                acc = acc + u * v
            out_v[pl.ds(p * _PASS + g * 16, 16)] = acc
            return carry

        lax.fori_loop(0, _PASS // 16, group_body, 0)

    pltpu.sync_copy(out_v, out_hbm.at[pl.ds(base, _BPW)])


def kernel(user_idx, item_idx, user_emb, item_emb):
    mesh = plsc.VectorSubcoreMesh(core_axis_name="c", subcore_axis_name="s")
    f = pl.kernel(
        _bpr_body,
        out_type=jax.ShapeDtypeStruct((_B,), jnp.float32),
        mesh=mesh,
        compiler_params=pltpu.CompilerParams(needs_layout_passes=False),
        scratch_types=[
            pltpu.VMEM((_BPW,), jnp.int32),
            pltpu.VMEM((_BPW,), jnp.int32),
            pltpu.VMEM((_PASS * 8, _D), jnp.float32),
            pltpu.VMEM((_PASS * 8, _D), jnp.float32),
            pltpu.VMEM((_BPW,), jnp.float32),
            pltpu.SemaphoreType.DMA,
            pltpu.SemaphoreType.DMA,
        ],
    )
    return f(user_idx, item_idx, user_emb, item_emb)
